# baseline (speedup 1.0000x reference)
import functools

import jax
import jax.numpy as jnp
from jax import lax
from jax.experimental import pallas as pl
from jax.experimental.pallas import tpu as pltpu
from jax.experimental.pallas import tpu_sc as plsc

NUM_BUCKETS = 32
NUM_HEADS = 16
Q_LEN = 2048
K_LEN = 2048
DIAG = Q_LEN + K_LEN
NSHIFT = 8
LANES = 16
NRING = 4


_TAKE_DNUMS = lax.GatherDimensionNumbers(
    offset_dims=(), collapsed_slice_dims=(0,), start_index_map=(0,)
)


def _take(v, idx):
    return lax.gather(
        v,
        idx[:, None],
        _TAKE_DNUMS,
        (1,),
        mode=lax.GatherScatterMode.PROMISE_IN_BOUNDS,
    )


def _sc_body(w_hbm, out_hbm, w_v, dvec, dvec8, stg, sem_out):
    c = lax.axis_index("c")
    s = lax.axis_index("s")
    h = s
    lane = lax.iota(jnp.int32, LANES)

    pltpu.sync_copy(w_hbm, w_v)
    h_vec = jnp.broadcast_to(h, (LANES,)).astype(jnp.int32)
    lo = jnp.zeros((LANES,), jnp.float32)
    hi = jnp.zeros((LANES,), jnp.float32)
    for b in range(NUM_BUCKETS):
        wb = _take(w_v[pl.ds(b * NUM_HEADS, LANES)], h_vec)
        if b < LANES:
            lo = jnp.where(lane == b, wb, lo)
        else:
            hi = jnp.where(lane == b - LANES, wb, hi)

    @plsc.parallel_loop(0, DIAG // LANES, unroll=4)
    def diag_step(t):
        j = lane + t * LANES
        d = j - (K_LEN - 1)
        n = -d
        isneg = n < 0
        na = jnp.abs(n)
        issmall = na < 8
        nc = jnp.maximum(na, 1)
        sq = (nc * nc).astype(jnp.float32)
        e = (lax.bitcast_convert_type(sq, jnp.int32) >> 23) - 127
        large = jnp.minimum(8 + (e - 6), 15)
        b = jnp.where(issmall, na, large)
        b = jnp.where(isneg, b + LANES, b)
        val = jnp.where(b < LANES, _take(lo, b & 15), _take(hi, b & 15))
        dvec[pl.ds(t * LANES, LANES)] = val

    for r in range(NSHIFT):
        if r == 0:

            @plsc.parallel_loop(0, DIAG // LANES, unroll=4)
            def shift0_step(t):
                dvec8[pl.ds(t * LANES, LANES)] = dvec[pl.ds(t * LANES, LANES)]

        else:
            idx = (lane + r) & 15

            @plsc.parallel_loop(0, DIAG // LANES, unroll=4)
            def shift_step(t, r=r, idx=idx):
                v = dvec[pl.ds(t * LANES, LANES)]
                v2 = dvec[pl.ds(t * LANES + LANES, LANES)]
                out = jnp.where(lane < LANES - r, _take(v, idx), _take(v2, idx))
                dvec8[pl.ds(r * DIAG + t * LANES, LANES)] = out

    def _wait_ship():
        pltpu.make_async_copy(
            stg.at[0], out_hbm.at[0, 0, pl.ds(0, 8), :], sem_out
        ).wait()

    def grp_step(g, carry):
        b8 = pl.multiple_of((255 - c * 128 - g) * 8, 8)
        stg_g = stg.at[lax.rem(g, NRING)]

        @pl.when(g >= NRING)
        def _wait_one():
            _wait_ship()

        @plsc.parallel_loop(0, K_LEN // LANES, unroll=4)
        def cp(v):
            col = pl.multiple_of(v * LANES, LANES)
            for r in range(8):
                stg_g[r, pl.ds(col, LANES)] = dvec8[
                    pl.ds(b8 + (7 - r) * DIAG + col, LANES)
                ]

        row8 = pl.multiple_of((c * 128 + g) * 8, 8)
        pltpu.async_copy(stg_g, out_hbm.at[0, h, pl.ds(row8, 8), :], sem_out)
        return carry

    lax.fori_loop(0, Q_LEN // 2 // 8, grp_step, 0)

    def drain_step(i, carry):
        _wait_ship()
        return carry

    lax.fori_loop(0, NRING, drain_step, 0)


def kernel(q_len, k_len, relative_attention_bias):
    mesh = plsc.VectorSubcoreMesh(core_axis_name="c", subcore_axis_name="s")
    run = functools.partial(
        pl.kernel,
        mesh=mesh,
        out_type=jax.ShapeDtypeStruct((1, NUM_HEADS, Q_LEN, K_LEN), jnp.float32),
        scratch_types=[
            pltpu.VMEM((NUM_BUCKETS * NUM_HEADS,), jnp.float32),
            pltpu.VMEM((DIAG + LANES,), jnp.float32),
            pltpu.VMEM((NSHIFT * DIAG,), jnp.float32),
            pltpu.VMEM((NRING, 8, K_LEN), jnp.float32),
            pltpu.SemaphoreType.DMA,
        ],
    )(_sc_body)
    return run(relative_attention_bias.reshape(NUM_BUCKETS * NUM_HEADS))


# speedup vs baseline: 1.0127x; 1.0127x over previous
import functools

import jax
import jax.numpy as jnp
from jax import lax
from jax.experimental import pallas as pl
from jax.experimental.pallas import tpu as pltpu
from jax.experimental.pallas import tpu_sc as plsc

NUM_BUCKETS = 32
NUM_HEADS = 16
Q_LEN = 2048
K_LEN = 2048
DIAG = Q_LEN + K_LEN
NSHIFT = 8
LANES = 16


_TAKE_DNUMS = lax.GatherDimensionNumbers(
    offset_dims=(), collapsed_slice_dims=(0,), start_index_map=(0,)
)


def _take(v, idx):
    return lax.gather(
        v,
        idx[:, None],
        _TAKE_DNUMS,
        (1,),
        mode=lax.GatherScatterMode.PROMISE_IN_BOUNDS,
    )


def _sc_body(w_hbm, out_hbm, w_v, dvec, dvec8, stg_a, stg_b, sem_out):
    c = lax.axis_index("c")
    s = lax.axis_index("s")
    h = s
    lane = lax.iota(jnp.int32, LANES)

    pltpu.sync_copy(w_hbm, w_v)
    h_vec = jnp.broadcast_to(h, (LANES,)).astype(jnp.int32)
    lo = jnp.zeros((LANES,), jnp.float32)
    hi = jnp.zeros((LANES,), jnp.float32)
    for b in range(NUM_BUCKETS):
        wb = _take(w_v[pl.ds(b * NUM_HEADS, LANES)], h_vec)
        if b < LANES:
            lo = jnp.where(lane == b, wb, lo)
        else:
            hi = jnp.where(lane == b - LANES, wb, hi)

    @plsc.parallel_loop(0, DIAG // LANES, unroll=4)
    def diag_step(t):
        j = lane + t * LANES
        d = j - (K_LEN - 1)
        n = -d
        isneg = n < 0
        na = jnp.abs(n)
        issmall = na < 8
        nc = jnp.maximum(na, 1)
        sq = (nc * nc).astype(jnp.float32)
        e = (lax.bitcast_convert_type(sq, jnp.int32) >> 23) - 127
        large = jnp.minimum(8 + (e - 6), 15)
        b = jnp.where(issmall, na, large)
        b = jnp.where(isneg, b + LANES, b)
        val = jnp.where(b < LANES, _take(lo, b & 15), _take(hi, b & 15))
        dvec[pl.ds(t * LANES, LANES)] = val

    for r in range(NSHIFT):
        if r == 0:

            @plsc.parallel_loop(0, DIAG // LANES, unroll=4)
            def shift0_step(t):
                dvec8[pl.ds(t * LANES, LANES)] = dvec[pl.ds(t * LANES, LANES)]

        else:
            idx = (lane + r) & 15

            @plsc.parallel_loop(0, DIAG // LANES, unroll=4)
            def shift_step(t, r=r, idx=idx):
                v = dvec[pl.ds(t * LANES, LANES)]
                v2 = dvec[pl.ds(t * LANES + LANES, LANES)]
                out = jnp.where(lane < LANES - r, _take(v, idx), _take(v2, idx))
                dvec8[pl.ds(r * DIAG + t * LANES, LANES)] = out

    def _build(stg, b8):
        @plsc.parallel_loop(0, K_LEN // LANES, unroll=8)
        def cp(v):
            col = pl.multiple_of(v * LANES, LANES)
            for r in range(8):
                stg[r, pl.ds(col, LANES)] = dvec8[
                    pl.ds(b8 + (7 - r) * DIAG + col, LANES)
                ]

    def _ship(stg, g):
        row8 = pl.multiple_of((c * 128 + g) * 8, 8)
        pltpu.async_copy(stg, out_hbm.at[0, h, pl.ds(row8, 8), :], sem_out)

    def _wait_ship():
        pltpu.make_async_copy(
            stg_a, out_hbm.at[0, 0, pl.ds(0, 8), :], sem_out
        ).wait()

    def grp_step(g, carry):
        b8 = pl.multiple_of((255 - c * 128 - g) * 8, 8)

        @pl.when(g >= 2)
        def _wait_one():
            _wait_ship()

        @pl.when(lax.rem(g, 2) == 0)
        def _even():
            _build(stg_a, b8)
            _ship(stg_a, g)

        @pl.when(lax.rem(g, 2) == 1)
        def _odd():
            _build(stg_b, b8)
            _ship(stg_b, g)

        return carry

    lax.fori_loop(0, Q_LEN // 2 // 8, grp_step, 0)

    def drain_step(i, carry):
        _wait_ship()
        return carry

    lax.fori_loop(0, 2, drain_step, 0)


def kernel(q_len, k_len, relative_attention_bias):
    mesh = plsc.VectorSubcoreMesh(core_axis_name="c", subcore_axis_name="s")
    run = functools.partial(
        pl.kernel,
        mesh=mesh,
        out_type=jax.ShapeDtypeStruct((1, NUM_HEADS, Q_LEN, K_LEN), jnp.float32),
        scratch_types=[
            pltpu.VMEM((NUM_BUCKETS * NUM_HEADS,), jnp.float32),
            pltpu.VMEM((DIAG + LANES,), jnp.float32),
            pltpu.VMEM((NSHIFT * DIAG,), jnp.float32),
            pltpu.VMEM((8, K_LEN), jnp.float32),
            pltpu.VMEM((8, K_LEN), jnp.float32),
            pltpu.SemaphoreType.DMA,
        ],
    )(_sc_body)
    return run(relative_attention_bias.reshape(NUM_BUCKETS * NUM_HEADS))
